# baseline (speedup 1.0000x reference)
import functools

import jax
import jax.numpy as jnp
from jax import lax
from jax.experimental import pallas as pl
from jax.experimental.pallas import tpu as pltpu
from jax.experimental.pallas import tpu_sc as plsc

N = 10000
H = 512
OUT = 256
E = 160000
C = 4
CW = H // C
NS = 16
NC = 2
EPT = E // NS
B = 128
EPTP = 10240
NB = EPTP // B
G = 40
NA = N + 8
RZ = 624
TAIL = N - NS * RZ


def _agg_body(tab_hbm, src_hbm, dst_hbm, zeros_hbm, sums_hbm,
              idx_s, idx_d, rows, acc, sem_a, sem_b):
    c = lax.axis_index("c")
    s = lax.axis_index("s")

    def fire(j, buf, sem):
        pltpu.async_copy(tab_hbm.at[idx_s.at[j]], rows.at[buf], sem)

    def drain(buf, sem):
        pltpu.make_async_copy(tab_hbm.at[pl.ds(0, B)], rows.at[buf],
                              sem).wait()

    for ch in range(C // NC):
        chunk = c * (C // NC) + ch
        pltpu.sync_copy(zeros_hbm, acc.at[pl.ds(s * RZ, RZ)])

        @pl.when(s == NS - 1)
        def _():
            pltpu.sync_copy(zeros_hbm.at[pl.ds(0, TAIL)],
                            acc.at[pl.ds(NS * RZ, TAIL)])

        plsc.subcore_barrier()

        for g in range(NB // G):
            pltpu.sync_copy(src_hbm.at[chunk, s, pl.ds(g * G, G)], idx_s)
            pltpu.sync_copy(dst_hbm.at[s, pl.ds(g * G, G)], idx_d)

            fire(0, 0, sem_a)

            def body(i, carry):
                j0 = 2 * i
                j1 = 2 * i + 1
                fire(j1, 1, sem_b)
                drain(0, sem_a)
                pltpu.sync_copy(rows.at[0], acc.at[idx_d.at[j0]], add=True)

                @pl.when(j1 + 1 < G)
                def _():
                    fire(j1 + 1, 0, sem_a)

                drain(1, sem_b)
                pltpu.sync_copy(rows.at[1], acc.at[idx_d.at[j1]], add=True)
                return carry

            lax.fori_loop(0, G // 2, body, 0)

        plsc.subcore_barrier()
        pltpu.sync_copy(acc.at[pl.ds(s * RZ, RZ)],
                        sums_hbm.at[chunk, pl.ds(s * RZ, RZ)])

        @pl.when(s == NS - 1)
        def _():
            pltpu.sync_copy(acc.at[pl.ds(NS * RZ, TAIL)],
                            sums_hbm.at[chunk, pl.ds(NS * RZ, TAIL)])


@functools.lru_cache(maxsize=None)
def _make_agg():
    mesh = plsc.VectorSubcoreMesh(core_axis_name="c", subcore_axis_name="s", num_cores=NC, num_subcores=NS)

    return pl.kernel(
        _agg_body,
        out_type=jax.ShapeDtypeStruct((C, N, CW), jnp.float32),
        mesh=mesh,
        scratch_types=[
            pltpu.VMEM((G, B), jnp.int32),
            pltpu.VMEM((G, B), jnp.int32),
            pltpu.VMEM((2, B, CW), jnp.float32),
            pltpu.VMEM_SHARED((NA, CW), jnp.float32),
            pltpu.SemaphoreType.DMA,
            pltpu.SemaphoreType.DMA,
        ],
    )


def _counts_body(dw_hbm, dwb_hbm, ones_hbm, zeros_hbm, cw_hbm, cwb_hbm,
                 idx_d, ones_v, cacc, sem):
    c = lax.axis_index("c")
    s = lax.axis_index("s")
    pltpu.sync_copy(ones_hbm, ones_v)
    pltpu.sync_copy(zeros_hbm, cacc.at[pl.ds(s * RZ, RZ)])

    @pl.when(s == NS - 1)
    def _():
        pltpu.sync_copy(zeros_hbm.at[pl.ds(0, TAIL)],
                        cacc.at[pl.ds(NS * RZ, TAIL)])

    @pl.when(c == 0)
    def _():
        pltpu.sync_copy(dw_hbm.at[s], idx_d)

    @pl.when(c == 1)
    def _():
        pltpu.sync_copy(dwb_hbm.at[s], idx_d)

    plsc.subcore_barrier()

    def body(j, carry):
        pltpu.sync_copy(ones_v, cacc.at[idx_d.at[j]], add=True)
        return carry

    lax.fori_loop(0, NB, body, 0)
    plsc.subcore_barrier()

    @pl.when(c == 0)
    def _():
        pltpu.sync_copy(cacc.at[pl.ds(s * RZ, RZ)],
                        cw_hbm.at[pl.ds(s * RZ, RZ)])

        @pl.when(s == NS - 1)
        def _():
            pltpu.sync_copy(cacc.at[pl.ds(NS * RZ, TAIL)],
                            cw_hbm.at[pl.ds(NS * RZ, TAIL)])

    @pl.when(c == 1)
    def _():
        pltpu.sync_copy(cacc.at[pl.ds(s * RZ, RZ)],
                        cwb_hbm.at[pl.ds(s * RZ, RZ)])

        @pl.when(s == NS - 1)
        def _():
            pltpu.sync_copy(cacc.at[pl.ds(NS * RZ, TAIL)],
                            cwb_hbm.at[pl.ds(NS * RZ, TAIL)])


@functools.lru_cache(maxsize=None)
def _make_counts():
    mesh = plsc.VectorSubcoreMesh(core_axis_name="c", subcore_axis_name="s", num_cores=NC, num_subcores=NS)
    return pl.kernel(
        _counts_body,
        out_type=(jax.ShapeDtypeStruct((N, CW), jnp.float32),
                  jax.ShapeDtypeStruct((N, CW), jnp.float32)),
        mesh=mesh,
        scratch_types=[
            pltpu.VMEM((NB, B), jnp.int32),
            pltpu.VMEM((B, CW), jnp.float32),
            pltpu.VMEM_SHARED((NA, CW), jnp.float32),
            pltpu.SemaphoreType.DMA,
        ],
    )


_RB = 2000


def _proj_kernel(x_ref, w_ref, b_ref, out_ref):
    x = x_ref[...]
    for cc in range(C):
        out_ref[cc] = jnp.dot(x, w_ref[:, cc, :],
                              preferred_element_type=jnp.float32) + b_ref[cc]


def _proj(x, w, b):
    return pl.pallas_call(
        _proj_kernel,
        grid=(N // _RB,),
        in_specs=[
            pl.BlockSpec((_RB, H), lambda i: (i, 0)),
            pl.BlockSpec((H, C, CW), lambda i: (0, 0, 0)),
            pl.BlockSpec((C, CW), lambda i: (0, 0)),
        ],
        out_specs=pl.BlockSpec((C, _RB, CW), lambda i: (0, i, 0)),
        out_shape=jax.ShapeDtypeStruct((C, N, CW), jnp.float32),
    )(x, w, b)


def _sage_kernel(sums_ref, cnt_ref, x_ref, wl_ref, b_ref, wr_ref, out_ref):
    scale = 1.0 / jnp.maximum(cnt_ref[:, 0:1], 1.0)
    m = jnp.concatenate([sums_ref[k] * scale for k in range(C)], axis=1)
    xd = jnp.concatenate([x_ref[k] for k in range(C)], axis=1)
    r = (jnp.dot(m, wl_ref[...], preferred_element_type=jnp.float32)
         + jnp.dot(xd, wr_ref[...], preferred_element_type=jnp.float32)
         + b_ref[...])
    r = jnp.maximum(r, 0.0)
    for cc in range(C):
        out_ref[cc] = r[:, cc * CW:(cc + 1) * CW]


def _sage(sums, cnt, x, wl, b, wr):
    return pl.pallas_call(
        _sage_kernel,
        grid=(N // _RB,),
        in_specs=[
            pl.BlockSpec((C, _RB, CW), lambda i: (0, i, 0)),
            pl.BlockSpec((_RB, CW), lambda i: (i, 0)),
            pl.BlockSpec((C, _RB, CW), lambda i: (0, i, 0)),
            pl.BlockSpec((H, H), lambda i: (0, 0)),
            pl.BlockSpec((1, H), lambda i: (0, 0)),
            pl.BlockSpec((H, H), lambda i: (0, 0)),
        ],
        out_specs=pl.BlockSpec((C, _RB, CW), lambda i: (0, i, 0)),
        out_shape=jax.ShapeDtypeStruct((C, N, CW), jnp.float32),
    )(sums, cnt, x, wl, b, wr)


def _head_kernel(sums_ref, cnt_ref, p1_ref, wl_ref, bl_ref, wr_ref,
                 w1_ref, b1_ref, w2_ref, b2_ref, out_ref):
    scale = 1.0 / jnp.maximum(cnt_ref[:, 0:1], 1.0)
    m = jnp.concatenate([sums_ref[k] * scale for k in range(C)], axis=1)
    p1 = jnp.concatenate([p1_ref[k] for k in range(C)], axis=1)
    p2 = (jnp.dot(m, wl_ref[...], preferred_element_type=jnp.float32)
          + jnp.dot(p1, wr_ref[...], preferred_element_type=jnp.float32)
          + bl_ref[...])
    p2 = jnp.maximum(p2, 0.0)
    h = jnp.maximum(jnp.dot(p2, w1_ref[...],
                            preferred_element_type=jnp.float32)
                    + b1_ref[...], 0.0)
    out_ref[...] = jnp.dot(h, w2_ref[...],
                           preferred_element_type=jnp.float32) + b2_ref[...]


def _head(sums, cnt, p1, wl, bl, wr, w1, b1, w2, b2):
    return pl.pallas_call(
        _head_kernel,
        grid=(N // _RB,),
        in_specs=[
            pl.BlockSpec((C, _RB, CW), lambda i: (0, i, 0)),
            pl.BlockSpec((_RB, CW), lambda i: (i, 0)),
            pl.BlockSpec((C, _RB, CW), lambda i: (0, i, 0)),
            pl.BlockSpec((H, H), lambda i: (0, 0)),
            pl.BlockSpec((1, H), lambda i: (0, 0)),
            pl.BlockSpec((H, H), lambda i: (0, 0)),
            pl.BlockSpec((H, H), lambda i: (0, 0)),
            pl.BlockSpec((1, H), lambda i: (0, 0)),
            pl.BlockSpec((H, OUT), lambda i: (0, 0)),
            pl.BlockSpec((1, OUT), lambda i: (0, 0)),
        ],
        out_specs=pl.BlockSpec((_RB, OUT), lambda i: (i, 0)),
        out_shape=jax.ShapeDtypeStruct((N, OUT), jnp.float32),
    )(sums, cnt, p1, wl, bl, wr, w1, b1, w2, b2)


@jax.jit
def kernel(x_author, edge_index_writes, edge_index_written_by, paper_emb,
           W_proj, b_proj, Wl1p, bl1p, Wr1p, Wl1a, bl1a, Wr1a,
           Wl2p, bl2p, Wr2p, W1, b1, W2, b2):
    f32 = jnp.float32
    src_w = edge_index_writes[0].astype(jnp.int32)
    dst_w = edge_index_writes[1].astype(jnp.int32)
    src_wb = edge_index_written_by[0].astype(jnp.int32)
    dst_wb = edge_index_written_by[1].astype(jnp.int32)

    pad = EPTP - EPT

    def prep_src(srci):
        t = jnp.pad(srci.reshape(NS, E // NS), ((0, 0), (0, pad)))
        off = (jnp.arange(C, dtype=jnp.int32) * N)[:, None, None]
        return (t[None] + off).reshape(C, NS, NB, B)

    def prep_dst(dsti):
        t = jnp.pad(dsti.reshape(NS, E // NS), ((0, 0), (0, pad)),
                    constant_values=N)
        return t.reshape(NS, NB, B)

    src_w4 = prep_src(src_w)
    src_wb4 = prep_src(src_wb)
    dst_w3 = prep_dst(dst_w)
    dst_wb3 = prep_dst(dst_wb)

    zeros_acc = jnp.zeros((RZ, CW), f32)
    ones_b = jnp.ones((B, CW), f32)

    p_c = paper_emb.reshape(N, C, CW).transpose(1, 0, 2)

    cnt_w, cnt_wb = _make_counts()(dst_w3, dst_wb3, ones_b, zeros_acc)

    a_c = _proj(x_author, W_proj.reshape(H, C, CW), b_proj.reshape(C, CW))

    s1a = _make_agg()(p_c.reshape(C * N, CW), src_wb4, dst_wb3, zeros_acc)
    a1 = _sage(s1a, cnt_wb, a_c, Wl1a, bl1a.reshape(1, H), Wr1a)

    s1p = _make_agg()(a_c.reshape(C * N, CW), src_w4, dst_w3, zeros_acc)
    p1 = _sage(s1p, cnt_w, p_c, Wl1p, bl1p.reshape(1, H), Wr1p)

    s2p = _make_agg()(a1.reshape(C * N, CW), src_w4, dst_w3, zeros_acc)

    return _head(s2p, cnt_w, p1, Wl2p, bl2p.reshape(1, H), Wr2p,
                 W1, b1.reshape(1, H), W2, b2.reshape(1, OUT))


# speedup vs baseline: 2.0163x; 2.0163x over previous
import functools

import jax
import jax.numpy as jnp
from jax import lax
from jax.experimental import pallas as pl
from jax.experimental.pallas import tpu as pltpu
from jax.experimental.pallas import tpu_sc as plsc

N = 10000
H = 512
OUT = 256
E = 160000
C = 4
CW = H // C
NS = 16
NC = 2
EPT = E // NS
B = 125
EPTP = 10000
NB = EPTP // B
G = 40
NA = N + 8
RZ = 624
TAIL = N - NS * RZ


def _agg_body(tab_hbm, src_hbm, dst_hbm, zeros_hbm, sums_hbm,
              idx_s, idx_d, rows, acc, sem_a, sem_b):
    c = lax.axis_index("c")
    s = lax.axis_index("s")

    def fire(j, buf, sem):
        pltpu.async_copy(tab_hbm.at[idx_s.at[j]], rows.at[buf], sem)

    def drain(j, buf, sem):
        pltpu.make_async_copy(tab_hbm.at[idx_s.at[j]], rows.at[buf],
                              sem).wait()

    for ch in range(C // NC):
        chunk = c * (C // NC) + ch
        pltpu.sync_copy(zeros_hbm, acc.at[pl.ds(s * RZ, RZ)])

        @pl.when(s == NS - 1)
        def _():
            pltpu.sync_copy(zeros_hbm.at[pl.ds(0, TAIL)],
                            acc.at[pl.ds(NS * RZ, TAIL)])

        plsc.subcore_barrier()

        for g in range(NB // G):
            pltpu.sync_copy(src_hbm.at[chunk, s, pl.ds(g * G, G)], idx_s)
            pltpu.sync_copy(dst_hbm.at[s, pl.ds(g * G, G)], idx_d)

            fire(0, 0, sem_a)

            def body(i, carry):
                j0 = 2 * i
                j1 = 2 * i + 1
                fire(j1, 1, sem_b)
                drain(j0, 0, sem_a)
                pltpu.sync_copy(rows.at[0], acc.at[idx_d.at[j0]], add=True)

                @pl.when(j1 + 1 < G)
                def _():
                    fire(j1 + 1, 0, sem_a)

                drain(j1, 1, sem_b)
                pltpu.sync_copy(rows.at[1], acc.at[idx_d.at[j1]], add=True)
                return carry

            lax.fori_loop(0, G // 2, body, 0)

        plsc.subcore_barrier()
        pltpu.sync_copy(acc.at[pl.ds(s * RZ, RZ)],
                        sums_hbm.at[chunk, pl.ds(s * RZ, RZ)])

        @pl.when(s == NS - 1)
        def _():
            pltpu.sync_copy(acc.at[pl.ds(NS * RZ, TAIL)],
                            sums_hbm.at[chunk, pl.ds(NS * RZ, TAIL)])


@functools.lru_cache(maxsize=None)
def _make_agg():
    mesh = plsc.VectorSubcoreMesh(core_axis_name="c", subcore_axis_name="s", num_cores=NC, num_subcores=NS)

    return pl.kernel(
        _agg_body,
        out_type=jax.ShapeDtypeStruct((C, N, CW), jnp.float32),
        mesh=mesh,
        scratch_types=[
            pltpu.VMEM((G, B), jnp.int32),
            pltpu.VMEM((G, B), jnp.int32),
            pltpu.VMEM((2, B, CW), jnp.float32),
            pltpu.VMEM_SHARED((NA, CW), jnp.float32),
            pltpu.SemaphoreType.DMA,
            pltpu.SemaphoreType.DMA,
        ],
    )


def _counts_body(dw_hbm, dwb_hbm, ones_hbm, zeros_hbm, cw_hbm, cwb_hbm,
                 idx_d, ones_v, cacc, sem):
    c = lax.axis_index("c")
    s = lax.axis_index("s")
    pltpu.sync_copy(ones_hbm, ones_v)
    pltpu.sync_copy(zeros_hbm, cacc.at[pl.ds(s * RZ, RZ)])

    @pl.when(s == NS - 1)
    def _():
        pltpu.sync_copy(zeros_hbm.at[pl.ds(0, TAIL)],
                        cacc.at[pl.ds(NS * RZ, TAIL)])

    @pl.when(c == 0)
    def _():
        pltpu.sync_copy(dw_hbm.at[s], idx_d)

    @pl.when(c == 1)
    def _():
        pltpu.sync_copy(dwb_hbm.at[s], idx_d)

    plsc.subcore_barrier()

    def body(j, carry):
        pltpu.sync_copy(ones_v, cacc.at[idx_d.at[j]], add=True)
        return carry

    lax.fori_loop(0, NB, body, 0)
    plsc.subcore_barrier()

    @pl.when(c == 0)
    def _():
        pltpu.sync_copy(cacc.at[pl.ds(s * RZ, RZ)],
                        cw_hbm.at[pl.ds(s * RZ, RZ)])

        @pl.when(s == NS - 1)
        def _():
            pltpu.sync_copy(cacc.at[pl.ds(NS * RZ, TAIL)],
                            cw_hbm.at[pl.ds(NS * RZ, TAIL)])

    @pl.when(c == 1)
    def _():
        pltpu.sync_copy(cacc.at[pl.ds(s * RZ, RZ)],
                        cwb_hbm.at[pl.ds(s * RZ, RZ)])

        @pl.when(s == NS - 1)
        def _():
            pltpu.sync_copy(cacc.at[pl.ds(NS * RZ, TAIL)],
                            cwb_hbm.at[pl.ds(NS * RZ, TAIL)])


@functools.lru_cache(maxsize=None)
def _make_counts():
    mesh = plsc.VectorSubcoreMesh(core_axis_name="c", subcore_axis_name="s", num_cores=NC, num_subcores=NS)
    return pl.kernel(
        _counts_body,
        out_type=(jax.ShapeDtypeStruct((N, CW), jnp.float32),
                  jax.ShapeDtypeStruct((N, CW), jnp.float32)),
        mesh=mesh,
        scratch_types=[
            pltpu.VMEM((NB, B), jnp.int32),
            pltpu.VMEM((B, CW), jnp.float32),
            pltpu.VMEM_SHARED((NA, CW), jnp.float32),
            pltpu.SemaphoreType.DMA,
        ],
    )


_RB = 2000


def _proj_kernel(x_ref, w_ref, b_ref, out_ref):
    x = x_ref[...]
    for cc in range(C):
        out_ref[cc] = jnp.dot(x, w_ref[:, cc, :],
                              preferred_element_type=jnp.float32) + b_ref[cc]


def _proj(x, w, b):
    return pl.pallas_call(
        _proj_kernel,
        grid=(N // _RB,),
        in_specs=[
            pl.BlockSpec((_RB, H), lambda i: (i, 0)),
            pl.BlockSpec((H, C, CW), lambda i: (0, 0, 0)),
            pl.BlockSpec((C, CW), lambda i: (0, 0)),
        ],
        out_specs=pl.BlockSpec((C, _RB, CW), lambda i: (0, i, 0)),
        out_shape=jax.ShapeDtypeStruct((C, N, CW), jnp.float32),
    )(x, w, b)


def _sage_kernel(sums_ref, cnt_ref, x_ref, wl_ref, b_ref, wr_ref, out_ref):
    scale = 1.0 / jnp.maximum(cnt_ref[:, 0:1], 1.0)
    m = jnp.concatenate([sums_ref[k] * scale for k in range(C)], axis=1)
    xd = jnp.concatenate([x_ref[k] for k in range(C)], axis=1)
    r = (jnp.dot(m, wl_ref[...], preferred_element_type=jnp.float32)
         + jnp.dot(xd, wr_ref[...], preferred_element_type=jnp.float32)
         + b_ref[...])
    r = jnp.maximum(r, 0.0)
    for cc in range(C):
        out_ref[cc] = r[:, cc * CW:(cc + 1) * CW]


def _sage(sums, cnt, x, wl, b, wr):
    return pl.pallas_call(
        _sage_kernel,
        grid=(N // _RB,),
        in_specs=[
            pl.BlockSpec((C, _RB, CW), lambda i: (0, i, 0)),
            pl.BlockSpec((_RB, CW), lambda i: (i, 0)),
            pl.BlockSpec((C, _RB, CW), lambda i: (0, i, 0)),
            pl.BlockSpec((H, H), lambda i: (0, 0)),
            pl.BlockSpec((1, H), lambda i: (0, 0)),
            pl.BlockSpec((H, H), lambda i: (0, 0)),
        ],
        out_specs=pl.BlockSpec((C, _RB, CW), lambda i: (0, i, 0)),
        out_shape=jax.ShapeDtypeStruct((C, N, CW), jnp.float32),
    )(sums, cnt, x, wl, b, wr)


def _head_kernel(sums_ref, cnt_ref, p1_ref, wl_ref, bl_ref, wr_ref,
                 w1_ref, b1_ref, w2_ref, b2_ref, out_ref):
    scale = 1.0 / jnp.maximum(cnt_ref[:, 0:1], 1.0)
    m = jnp.concatenate([sums_ref[k] * scale for k in range(C)], axis=1)
    p1 = jnp.concatenate([p1_ref[k] for k in range(C)], axis=1)
    p2 = (jnp.dot(m, wl_ref[...], preferred_element_type=jnp.float32)
          + jnp.dot(p1, wr_ref[...], preferred_element_type=jnp.float32)
          + bl_ref[...])
    p2 = jnp.maximum(p2, 0.0)
    h = jnp.maximum(jnp.dot(p2, w1_ref[...],
                            preferred_element_type=jnp.float32)
                    + b1_ref[...], 0.0)
    out_ref[...] = jnp.dot(h, w2_ref[...],
                           preferred_element_type=jnp.float32) + b2_ref[...]


def _head(sums, cnt, p1, wl, bl, wr, w1, b1, w2, b2):
    return pl.pallas_call(
        _head_kernel,
        grid=(N // _RB,),
        in_specs=[
            pl.BlockSpec((C, _RB, CW), lambda i: (0, i, 0)),
            pl.BlockSpec((_RB, CW), lambda i: (i, 0)),
            pl.BlockSpec((C, _RB, CW), lambda i: (0, i, 0)),
            pl.BlockSpec((H, H), lambda i: (0, 0)),
            pl.BlockSpec((1, H), lambda i: (0, 0)),
            pl.BlockSpec((H, H), lambda i: (0, 0)),
            pl.BlockSpec((H, H), lambda i: (0, 0)),
            pl.BlockSpec((1, H), lambda i: (0, 0)),
            pl.BlockSpec((H, OUT), lambda i: (0, 0)),
            pl.BlockSpec((1, OUT), lambda i: (0, 0)),
        ],
        out_specs=pl.BlockSpec((_RB, OUT), lambda i: (i, 0)),
        out_shape=jax.ShapeDtypeStruct((N, OUT), jnp.float32),
    )(sums, cnt, p1, wl, bl, wr, w1, b1, w2, b2)


@jax.jit
def kernel(x_author, edge_index_writes, edge_index_written_by, paper_emb,
           W_proj, b_proj, Wl1p, bl1p, Wr1p, Wl1a, bl1a, Wr1a,
           Wl2p, bl2p, Wr2p, W1, b1, W2, b2):
    f32 = jnp.float32
    src_w = edge_index_writes[0].astype(jnp.int32)
    dst_w = edge_index_writes[1].astype(jnp.int32)
    src_wb = edge_index_written_by[0].astype(jnp.int32)
    dst_wb = edge_index_written_by[1].astype(jnp.int32)

    def prep_src(srci):
        t = srci.reshape(NS, EPT)
        off = (jnp.arange(C, dtype=jnp.int32) * N)[:, None, None]
        return (t[None] + off).reshape(C, NS, NB, B)

    def prep_dst(dsti):
        return dsti.reshape(NS, NB, B)

    src_w4 = prep_src(src_w)
    src_wb4 = prep_src(src_wb)
    dst_w3 = prep_dst(dst_w)
    dst_wb3 = prep_dst(dst_wb)

    zeros_acc = jnp.zeros((RZ, CW), f32)
    ones_b = jnp.ones((B, CW), f32)

    p_c = paper_emb.reshape(N, C, CW).transpose(1, 0, 2)

    cnt_w, cnt_wb = _make_counts()(dst_w3, dst_wb3, ones_b, zeros_acc)

    a_c = _proj(x_author, W_proj.reshape(H, C, CW), b_proj.reshape(C, CW))

    s1a = _make_agg()(p_c.reshape(C * N, CW), src_wb4, dst_wb3, zeros_acc)
    a1 = _sage(s1a, cnt_wb, a_c, Wl1a, bl1a.reshape(1, H), Wr1a)

    s1p = _make_agg()(a_c.reshape(C * N, CW), src_w4, dst_w3, zeros_acc)
    p1 = _sage(s1p, cnt_w, p_c, Wl1p, bl1p.reshape(1, H), Wr1p)

    s2p = _make_agg()(a1.reshape(C * N, CW), src_w4, dst_w3, zeros_acc)

    return _head(s2p, cnt_w, p1, Wl2p, bl2p.reshape(1, H), Wr2p,
                 W1, b1.reshape(1, H), W2, b2.reshape(1, OUT))
